# Initial kernel scaffold; baseline (speedup 1.0000x reference)
import jax
import jax.numpy as jnp
from jax.experimental import pallas as pl


def kernel(n, rel_bias_table):
    raise NotImplementedError("write your pallas kernel here")


# speedup vs baseline: 144.1811x; 144.1811x over previous
import math

import jax
import jax.numpy as jnp
from jax.experimental import pallas as pl
from jax.experimental.pallas import tpu as pltpu

_NB = 32
_H = 16
_N = 2048
_BQ = 256
_DW = 4224
_LOG_DENOM = math.log(128 / 8)


def _diag_values(table_ref):
    j = jax.lax.broadcasted_iota(jnp.int32, (1, _DW), 1)
    rel = j - (_N - 1)
    neg = -rel
    res = jnp.where(neg < 0, _NB // 2, 0).astype(jnp.int32)
    na = jnp.abs(neg)
    is_small = na < 8
    n_safe = jnp.maximum(na, 1).astype(jnp.float32)
    vil = 8 + (jnp.log(n_safe / 8) / _LOG_DENOM * 8).astype(jnp.int32)
    vil = jnp.minimum(vil, 15)
    bucket = res + jnp.where(is_small, na, vil)
    acc = jnp.zeros((1, _DW), jnp.float32)
    for b in range(_NB):
        acc = jnp.where(bucket == b, table_ref[0, 0, b], acc)
    return acc


def _bias_body(table_ref, out_ref, shear_ref):
    qb = pl.program_id(1)

    @pl.when(qb == 0)
    def _build_shear():
        diag = _diag_values(table_ref)
        rep = jnp.broadcast_to(diag, (128, _DW))
        shear_ref[...] = pltpu.roll(rep, 1, 1, stride=1, stride_axis=0)

    base_t = qb * (_BQ // 128)
    for c in range(_BQ // 128):
        a = pl.multiple_of(2048 - 128 * (base_t + c), 128)
        out_ref[0, pl.ds(c * 128, 128), :] = shear_ref[:, pl.ds(a, _N)]


@jax.jit
def _rpb(table_t):
    return pl.pallas_call(
        _bias_body,
        grid=(_H, _N // _BQ),
        in_specs=[pl.BlockSpec((1, 1, _NB), lambda h, qb: (h, 0, 0))],
        out_specs=pl.BlockSpec((1, _BQ, _N), lambda h, qb: (h, qb, 0)),
        out_shape=jax.ShapeDtypeStruct((_H, _N, _N), jnp.float32),
        scratch_shapes=[pltpu.VMEM((128, _DW), jnp.float32)],
        compiler_params=pltpu.CompilerParams(
            dimension_semantics=("arbitrary", "arbitrary")),
    )(table_t)


def kernel(n, rel_bias_table):
    del n
    table_t = rel_bias_table.T.reshape(_H, 1, _NB)
    return _rpb(table_t)


# speedup vs baseline: 194.3229x; 1.3478x over previous
import math

import jax
import jax.numpy as jnp
from jax.experimental import pallas as pl
from jax.experimental.pallas import tpu as pltpu

_NB = 32
_H = 16
_N = 2048
_DW = 4224
_NT = _N // 128
_LOG_DENOM = math.log(128 / 8)


def _diag_values(table_ref, h):
    j = jax.lax.broadcasted_iota(jnp.int32, (1, _DW), 1)
    rel = j - (_N - 1)
    neg = -rel
    res = jnp.where(neg < 0, _NB // 2, 0).astype(jnp.int32)
    na = jnp.abs(neg)
    is_small = na < 8
    n_safe = jnp.maximum(na, 1).astype(jnp.float32)
    vil = 8 + (jnp.log(n_safe / 8) / _LOG_DENOM * 8).astype(jnp.int32)
    vil = jnp.minimum(vil, 15)
    bucket = res + jnp.where(is_small, na, vil)
    acc = jnp.zeros((1, _DW), jnp.float32)
    for b in range(_NB):
        acc = jnp.where(bucket == b, table_ref[h, 0, b], acc)
    return acc


def _chunk_copy(shear_ref, out_ref, h, t, sem):
    a = 2048 - 128 * t
    return pltpu.make_async_copy(
        shear_ref.at[:, pl.ds(a, _N)],
        out_ref.at[h, pl.ds(128 * t, 128), :],
        sem,
    )


def _bias_body(table_ref, out_ref, shear0, shear1, sem0, sem1):
    shears = (shear0, shear1)
    sems = (sem0, sem1)
    for h in range(_H):
        sh, sem = shears[h % 2], sems[h % 2]
        if h >= 2:
            for t in range(_NT):
                _chunk_copy(sh, out_ref, h - 2, t, sem).wait()
        diag = _diag_values(table_ref, h)
        rep = jnp.broadcast_to(diag, (128, _DW))
        sh[...] = pltpu.roll(rep, 1, 1, stride=1, stride_axis=0)
        for t in range(_NT):
            _chunk_copy(sh, out_ref, h, t, sem).start()
    for h in (_H - 2, _H - 1):
        for t in range(_NT):
            _chunk_copy(shears[h % 2], out_ref, h, t, sems[h % 2]).wait()


@jax.jit
def _rpb(table_t):
    return pl.pallas_call(
        _bias_body,
        in_specs=[pl.BlockSpec(memory_space=pltpu.VMEM)],
        out_specs=pl.BlockSpec(memory_space=pl.ANY),
        out_shape=jax.ShapeDtypeStruct((_H, _N, _N), jnp.float32),
        scratch_shapes=[
            pltpu.VMEM((128, _DW), jnp.float32),
            pltpu.VMEM((128, _DW), jnp.float32),
            pltpu.SemaphoreType.DMA,
            pltpu.SemaphoreType.DMA,
        ],
    )(table_t)


def kernel(n, rel_bias_table):
    del n
    table_t = rel_bias_table.T.reshape(_H, 1, _NB)
    return _rpb(table_t)
